# baseline (speedup 1.0000x reference)
import functools

import jax
import jax.numpy as jnp
from jax.experimental import pallas as pl
from jax.experimental.pallas import tpu as pltpu

MEM_ROWS = 100000
BLOCK_ROWS = 2000
VDIM = 512


def _soft_read_kernel(x_ref, fz_ref, k_ref, v_ref, o_ref, acc_ref, den_ref):
    i = pl.program_id(0)
    nblocks = pl.num_programs(0)

    x = x_ref[...]
    k = k_ref[...]
    v = v_ref[...]

    x_norm = jnp.sqrt(jnp.sum(x * x))
    num = jnp.sum(k * x, axis=1, keepdims=True)
    sq = jnp.sum(k * k, axis=1, keepdims=True)
    denom = jnp.maximum(x_norm * jnp.sqrt(sq), 1e-6)
    p = jnp.exp(num / denom)

    part = jnp.sum(p * v, axis=0, keepdims=True)
    psum = jnp.sum(p)

    @pl.when(i == 0)
    def _init():
        acc_ref[...] = jnp.zeros_like(acc_ref)
        den_ref[0, 0] = 0.0

    acc_ref[...] += part
    den_ref[0, 0] += psum

    @pl.when(i == nblocks - 1)
    def _finish():
        fz = fz_ref[...]
        o_ref[...] = 0.7 * (acc_ref[...] / den_ref[0, 0]) + 0.3 * fz


@functools.partial(jax.jit, static_argnames=("block_rows",))
def _soft_read(x_key, f_z_value, key_memory, value_memory, block_rows=BLOCK_ROWS):
    m, kdim = key_memory.shape
    v2d = value_memory.reshape(m, VDIM)
    fz2d = f_z_value.reshape(1, VDIM)
    nblocks = m // block_rows

    out = pl.pallas_call(
        _soft_read_kernel,
        grid=(nblocks,),
        in_specs=[
            pl.BlockSpec((1, kdim), lambda i: (0, 0)),
            pl.BlockSpec((1, VDIM), lambda i: (0, 0)),
            pl.BlockSpec((block_rows, kdim), lambda i: (i, 0)),
            pl.BlockSpec((block_rows, VDIM), lambda i: (i, 0)),
        ],
        out_specs=pl.BlockSpec((1, VDIM), lambda i: (0, 0)),
        out_shape=jax.ShapeDtypeStruct((1, VDIM), jnp.float32),
        scratch_shapes=[
            pltpu.VMEM((1, VDIM), jnp.float32),
            pltpu.SMEM((1, 1), jnp.float32),
        ],
    )(x_key, fz2d, key_memory, v2d)
    return out.reshape(f_z_value.shape)


def kernel(x_key, f_z_value, key_memory, value_memory):
    return _soft_read(x_key, f_z_value, key_memory, value_memory)


# speedup vs baseline: 1.0293x; 1.0293x over previous
import functools

import jax
import jax.numpy as jnp
from jax.experimental import pallas as pl
from jax.experimental.pallas import tpu as pltpu

MEM_ROWS = 100000
BLOCK_ROWS = 2000
VDIM = 512


def _soft_read_kernel(x_ref, fz_ref, k_ref, v_ref, o_ref, acc_ref, den_ref):
    i = pl.program_id(0)
    nblocks = pl.num_programs(0)

    x = x_ref[...]
    k = k_ref[...]
    v = v_ref[...]

    kt = k.T
    x_norm = jnp.sqrt(jnp.sum(x * x))
    num = jnp.dot(x, kt)
    ones = jnp.ones((1, 128), jnp.float32)
    sq = jnp.dot(ones, kt * kt)
    denom = jnp.maximum(x_norm * jnp.sqrt(sq), 1e-6)
    p = jnp.exp(num / denom)

    part = jnp.dot(p, v)
    psum = jnp.sum(p)

    @pl.when(i == 0)
    def _init():
        acc_ref[...] = jnp.zeros_like(acc_ref)
        den_ref[0, 0] = 0.0

    acc_ref[...] += part
    den_ref[0, 0] += psum

    @pl.when(i == nblocks - 1)
    def _finish():
        fz = fz_ref[...]
        o_ref[...] = 0.7 * (acc_ref[...] / den_ref[0, 0]) + 0.3 * fz


@functools.partial(jax.jit, static_argnames=("block_rows",))
def _soft_read(x_key, f_z_value, key_memory, value_memory, block_rows=BLOCK_ROWS):
    m, kdim = key_memory.shape
    v2d = value_memory.reshape(m, VDIM)
    fz2d = f_z_value.reshape(1, VDIM)
    nblocks = m // block_rows

    out = pl.pallas_call(
        _soft_read_kernel,
        grid=(nblocks,),
        in_specs=[
            pl.BlockSpec((1, kdim), lambda i: (0, 0)),
            pl.BlockSpec((1, VDIM), lambda i: (0, 0)),
            pl.BlockSpec((block_rows, kdim), lambda i: (i, 0)),
            pl.BlockSpec((block_rows, VDIM), lambda i: (i, 0)),
        ],
        out_specs=pl.BlockSpec((1, VDIM), lambda i: (0, 0)),
        out_shape=jax.ShapeDtypeStruct((1, VDIM), jnp.float32),
        scratch_shapes=[
            pltpu.VMEM((1, VDIM), jnp.float32),
            pltpu.SMEM((1, 1), jnp.float32),
        ],
    )(x_key, fz2d, key_memory, v2d)
    return out.reshape(f_z_value.shape)


def kernel(x_key, f_z_value, key_memory, value_memory):
    return _soft_read(x_key, f_z_value, key_memory, value_memory)


# speedup vs baseline: 1.0719x; 1.0414x over previous
import functools

import jax
import jax.numpy as jnp
from jax.experimental import pallas as pl
from jax.experimental.pallas import tpu as pltpu

MEM_ROWS = 100000
BLOCK_ROWS = 5000
VDIM = 512


def _soft_read_kernel(x_ref, fz_ref, k_ref, v_ref, o_ref, acc_ref, den_ref):
    i = pl.program_id(0)
    nblocks = pl.num_programs(0)

    x = x_ref[...]
    k = k_ref[...]
    v = v_ref[...]

    kt = k.T
    x_norm = jnp.sqrt(jnp.sum(x * x))
    num = jnp.dot(x, kt)
    ones = jnp.ones((1, 128), jnp.float32)
    sq = jnp.dot(ones, kt * kt)
    denom = jnp.maximum(x_norm * jnp.sqrt(sq), 1e-6)
    p = jnp.exp(num / denom)

    part = jnp.dot(p, v)
    psum = jnp.sum(p)

    @pl.when(i == 0)
    def _init():
        acc_ref[...] = jnp.zeros_like(acc_ref)
        den_ref[0, 0] = 0.0

    acc_ref[...] += part
    den_ref[0, 0] += psum

    @pl.when(i == nblocks - 1)
    def _finish():
        fz = fz_ref[...]
        o_ref[...] = 0.7 * (acc_ref[...] / den_ref[0, 0]) + 0.3 * fz


@functools.partial(jax.jit, static_argnames=("block_rows",))
def _soft_read(x_key, f_z_value, key_memory, value_memory, block_rows=BLOCK_ROWS):
    m, kdim = key_memory.shape
    v2d = value_memory.reshape(m, VDIM)
    fz2d = f_z_value.reshape(1, VDIM)
    nblocks = m // block_rows

    out = pl.pallas_call(
        _soft_read_kernel,
        grid=(nblocks,),
        in_specs=[
            pl.BlockSpec((1, kdim), lambda i: (0, 0)),
            pl.BlockSpec((1, VDIM), lambda i: (0, 0)),
            pl.BlockSpec((block_rows, kdim), lambda i: (i, 0)),
            pl.BlockSpec((block_rows, VDIM), lambda i: (i, 0)),
        ],
        out_specs=pl.BlockSpec((1, VDIM), lambda i: (0, 0)),
        out_shape=jax.ShapeDtypeStruct((1, VDIM), jnp.float32),
        scratch_shapes=[
            pltpu.VMEM((1, VDIM), jnp.float32),
            pltpu.SMEM((1, 1), jnp.float32),
        ],
    )(x_key, fz2d, key_memory, v2d)
    return out.reshape(f_z_value.shape)


def kernel(x_key, f_z_value, key_memory, value_memory):
    return _soft_read(x_key, f_z_value, key_memory, value_memory)
